# Initial kernel scaffold; baseline (speedup 1.0000x reference)
import jax
import jax.numpy as jnp
from jax.experimental import pallas as pl


def kernel(user, item, edge_index, Gu, Gi, Wconv, Wdense, Q):
    raise NotImplementedError("write your pallas kernel here")


# speedup vs baseline: 83.3923x; 83.3923x over previous
import functools

import jax
import jax.numpy as jnp
from jax import lax
from jax.experimental import pallas as pl
from jax.experimental.pallas import tpu as pltpu
from jax.experimental.pallas import tpu_sc as plsc

NU = 25000
NI = 25000
N = NU + NI
K = 64
KP = 128
B = 64
E = 800000
NC = 2
NS = 16
NW = NC * NS
L = 16

NP = 51200
NPR = NP // L
EPAD = 802816
EPT = EPAD // NW
ERT = EPT // L
NCH = 7
CR = ERT // NCH
ACC = 256
RPA = ACC // NS
DUMP = 2 * B

_CP = pltpu.CompilerParams(needs_layout_passes=False)
_MESH = plsc.VectorSubcoreMesh(
    core_axis_name="c", subcore_axis_name="s", num_cores=NC, num_subcores=NS
)


def _k1_body(col1, hist_out, colv, hist, sem):
    c = lax.axis_index("c")
    s = lax.axis_index("s")
    wid = c * NS + s
    cp = pltpu.async_copy(col1.at[pl.ds(wid * EPT, EPT)], colv, sem)

    def zero(i, carry):
        hist[pl.ds(i * L, L)] = jnp.zeros((L,), jnp.int32)
        return carry

    lax.fori_loop(0, NPR, zero, 0)
    cp.wait()

    def step(i, carry):
        col16 = colv[pl.ds(i * L, L)]
        cnt, lastm = plsc.scan_count(col16)
        plsc.addupdate_scatter(hist, [col16], cnt, mask=lastm)
        return carry

    lax.fori_loop(0, ERT, step, 0)
    pltpu.sync_copy(hist, hist_out.at[pl.ds(wid * NP, NP)])


_k1 = functools.partial(
    pl.kernel,
    out_type=jax.ShapeDtypeStruct((NW * NP,), jnp.int32),
    mesh=_MESH,
    compiler_params=_CP,
    scratch_types=[
        pltpu.VMEM((EPT,), jnp.int32),
        pltpu.VMEM((NP,), jnp.int32),
        pltpu.SemaphoreType.DMA,
    ],
)(_k1_body)


def _k2_body(hist_ref, dinv_ref):
    d = jnp.sum(hist_ref[...].astype(jnp.float32), axis=0)
    dinv_ref[...] = jnp.where(d > 0.0, lax.rsqrt(jnp.maximum(d, 1e-12)), 0.0)


def _k3_body(rowp1, colp1, dinv_h, x_hbm, tgt, acc_out, smap_out,
             dinv_v, T1, tvm, dtgt, rowv, colv, mrow, mslot, wtmp,
             idxst, rowst, xbuf, zstg, smv, acc_sh, sem):
    c = lax.axis_index("c")
    s = lax.axis_index("s")
    wid = c * NS + s
    lane = lax.iota(jnp.int32, L)
    zi16 = jnp.zeros((L,), jnp.int32)
    zf16 = jnp.zeros((L,), jnp.float32)

    cpd = pltpu.async_copy(dinv_h, dinv_v, sem)
    pltpu.sync_copy(tgt, tvm)

    def fill_t(i, carry):
        T1[pl.ds(i * L, L)] = zi16 - 1
        return carry

    lax.fori_loop(0, NPR, fill_t, 0)

    lane0 = lane == 0

    def set_t(b, carry):
        bi = zi16 + b
        tb = plsc.load_gather(tvm, [bi])
        plsc.store_scatter(T1, [tb], bi, mask=lane0)
        return carry

    lax.fori_loop(0, 2 * B, set_t, 0)
    cpd.wait()

    for g in range(2 * B // L):
        t16 = tvm[pl.ds(g * L, L)]
        dtgt[pl.ds(g * L, L)] = plsc.load_gather(dinv_v, [t16])
    dtgt[pl.ds(2 * B, L)] = zf16

    for i in range(RPA):
        for k_ in range(KP // L):
            zstg[i, pl.ds(k_ * L, L)] = zf16
    pltpu.sync_copy(zstg, acc_sh.at[pl.ds(s * RPA, RPA)])
    plsc.subcore_barrier()

    def chunk(ch, carry):
        eb = wid * EPT + ch * CR * L
        pltpu.sync_copy(rowp1.at[pl.ds(eb, CR * L)], rowv)
        pltpu.sync_copy(colp1.at[pl.ds(eb, CR * L)], colv)
        mcnt = jnp.int32(0)
        for i in range(CR):
            s16 = plsc.load_gather(T1, [colv[pl.ds(i * L, L)]])
            m = s16 >= 0
            plsc.store_compressed(mrow.at[pl.ds(mcnt, L)], rowv[pl.ds(i * L, L)], mask=m)
            plsc.store_compressed(mslot.at[pl.ds(mcnt, L)], s16, mask=m)
            mcnt = mcnt + jnp.sum(m.astype(jnp.int32))
        ngroups = (mcnt + (L - 1)) >> 4

        def grp(j, inner):
            b16 = j * L
            slots16 = mslot[pl.ds(b16, L)]
            rows16 = mrow[pl.ds(b16, L)]
            padm = (lane + b16) < mcnt
            slots_s = jnp.where(padm, slots16, DUMP)
            rows_s = jnp.where(padm, rows16, 0)
            drow = plsc.load_gather(dinv_v, [rows_s])
            dcol = plsc.load_gather(dtgt, [slots_s])
            wvec = drow * dcol
            rowst[...] = rows_s
            pltpu.async_copy(x_hbm.at[rowst], xbuf, sem).wait()
            for l in range(L):
                wl = jnp.take_along_axis(wvec, zi16 + l, axis=0)
                for cc in range(KP // L):
                    xbuf[l, pl.ds(cc * L, L)] = xbuf[l, pl.ds(cc * L, L)] * wl
            cnt16, _ = plsc.scan_count(slots_s)
            maxc = jnp.max(cnt16)

            def rnd(r, inner2):
                idxst[...] = jnp.where(cnt16 == r + 1, slots_s, DUMP)
                pltpu.sync_copy(xbuf, acc_sh.at[idxst], add=True)
                return inner2

            lax.fori_loop(0, maxc, rnd, 0)
            return inner

        lax.fori_loop(0, ngroups, grp, 0)
        return carry

    lax.fori_loop(0, NCH, chunk, 0)
    plsc.subcore_barrier()
    pltpu.sync_copy(acc_sh.at[pl.ds(s * RPA, RPA)], zstg)
    pltpu.sync_copy(zstg, acc_out.at[pl.ds(c * ACC + s * RPA, RPA)])

    @pl.when(wid == 0)
    def _():
        for g in range(2 * B // L):
            t16 = tvm[pl.ds(g * L, L)]
            smv[pl.ds(g * L, L)] = plsc.load_gather(T1, [t16])
        pltpu.sync_copy(smv, smap_out)


_k3 = functools.partial(
    pl.kernel,
    out_type=(
        jax.ShapeDtypeStruct((NC * ACC, KP), jnp.float32),
        jax.ShapeDtypeStruct((2 * B,), jnp.int32),
    ),
    mesh=_MESH,
    compiler_params=_CP,
    scratch_types=[
        pltpu.VMEM((NP,), jnp.float32),
        pltpu.VMEM((NP,), jnp.int32),
        pltpu.VMEM((2 * B,), jnp.int32),
        pltpu.VMEM((2 * B + L,), jnp.float32),
        pltpu.VMEM((CR * L,), jnp.int32),
        pltpu.VMEM((CR * L,), jnp.int32),
        pltpu.VMEM((CR * L + L,), jnp.int32),
        pltpu.VMEM((CR * L + L,), jnp.int32),
        pltpu.VMEM((L,), jnp.float32),
        pltpu.VMEM((L,), jnp.int32),
        pltpu.VMEM((L,), jnp.int32),
        pltpu.VMEM((L, KP), jnp.float32),
        pltpu.VMEM((RPA, KP), jnp.float32),
        pltpu.VMEM((2 * B,), jnp.int32),
        pltpu.VMEM_SHARED((ACC, KP), jnp.float32),
        pltpu.SemaphoreType.DMA,
    ],
)(_k3_body)


def _k4_body(acc_ref, sm_ref, wc_ref, wd_ref, q_ref, xui_ref, zu_ref, zi_ref):
    g = acc_ref[0] + acc_ref[1]
    g = g[: 2 * B]
    h = jnp.maximum(jnp.dot(g, wc_ref[...], preferred_element_type=jnp.float32), 0.0)
    h = jnp.maximum(jnp.dot(h, wd_ref[...], preferred_element_type=jnp.float32), 0.0)
    sm = sm_ref[...]
    P = (sm == lax.broadcasted_iota(jnp.int32, (2 * B, 2 * B), 1)).astype(jnp.float32)
    zeta = jnp.dot(P, h, preferred_element_type=jnp.float32)
    zu = zeta[:B]
    zi = zeta[B:]
    t1 = jnp.dot(q_ref[...], zi, preferred_element_type=jnp.float32)
    t2 = lax.dot_general(zu, t1, dimension_numbers=(((0,), (0,)), ((), ())),
                         preferred_element_type=jnp.float32)
    xui_ref[...] = 1.0 / (1.0 + jnp.exp(-t2))
    zu_ref[...] = zu
    zi_ref[...] = zi


def kernel(user, item, edge_index, Gu, Gi, Wconv, Wdense, Q):
    user = user.astype(jnp.int32)
    item = item.astype(jnp.int32)
    ei = edge_index.astype(jnp.int32)
    row = ei[0]
    col = ei[1]
    rowp = jnp.concatenate([row, jnp.zeros((EPAD - E,), jnp.int32)])
    colp = jnp.concatenate([col, jnp.full((EPAD - E,), NP - 1, jnp.int32)])

    targets = jnp.concatenate([user, item + NU])
    x = jnp.concatenate([Gu, Gi], axis=0)
    xpad = jnp.pad(x, ((0, 0), (0, KP - K)))

    hist = _k1(colp)
    dinv = pl.pallas_call(
        _k2_body,
        out_shape=jax.ShapeDtypeStruct((NP // 128, 128), jnp.float32),
    )(hist.reshape(NW, NP // 128, 128))
    acc1, smap = _k3(rowp, colp, dinv.reshape(NP), xpad, targets)
    wc_pad = jnp.pad(Wconv, ((0, KP - K), (0, 0)))
    xui, zu, zi = pl.pallas_call(
        _k4_body,
        out_shape=(
            jax.ShapeDtypeStruct((B, B), jnp.float32),
            jax.ShapeDtypeStruct((B, K), jnp.float32),
            jax.ShapeDtypeStruct((B, K), jnp.float32),
        ),
    )(acc1.reshape(NC, ACC, KP), smap.reshape(2 * B, 1), wc_pad, Wdense, Q)
    return (xui, zu, zi)
